# Initial kernel scaffold; baseline (speedup 1.0000x reference)
import jax
import jax.numpy as jnp
from jax.experimental import pallas as pl


def kernel(input_ids, attention_mask, init_workspace, emb_table):
    raise NotImplementedError("write your pallas kernel here")


# speedup vs baseline: 1.1396x; 1.1396x over previous
import functools

import jax
import jax.numpy as jnp
from jax import lax
from jax.experimental import pallas as pl
from jax.experimental.pallas import tpu as pltpu
from jax.experimental.pallas import tpu_sc as plsc

HIDDEN = 128
LANES = 16
NVREG = HIDDEN // LANES
NC, NS = 2, 16
NW = NC * NS


_GATHER_DNUMS = lax.GatherDimensionNumbers(
    offset_dims=(), collapsed_slice_dims=(0,), start_index_map=(0,))


def _permute(x, idx):
    return lax.gather(x, idx.reshape(LANES, 1), _GATHER_DNUMS, (1,),
                      mode=lax.GatherScatterMode.PROMISE_IN_BOUNDS)


def _hsum(x):
    lane = lax.iota(jnp.int32, LANES)
    for s in (8, 4, 2, 1):
        x = x + _permute(x, lane ^ s)
    return x


def _rsqrt(x):
    i = plsc.bitcast(x, jnp.int32)
    y = plsc.bitcast(jnp.int32(0x5F3759DF) - (i >> 1), jnp.float32)
    for _ in range(3):
        y = y * (1.5 - 0.5 * x * y * y)
    return y


def _normalize_row(ref, r):
    vs = [ref[r, pl.ds(j * LANES, LANES)] for j in range(NVREG)]
    acc = vs[0]
    for j in range(1, NVREG):
        acc = acc + vs[j]
    mean = _hsum(acc) * (1.0 / HIDDEN)
    ds = [v - mean for v in vs]
    acc2 = ds[0] * ds[0]
    for j in range(1, NVREG):
        acc2 = acc2 + ds[j] * ds[j]
    var = _hsum(acc2) * (1.0 / (HIDDEN - 1))
    rstd = _rsqrt(var)
    for j in range(NVREG):
        ref[r, pl.ds(j * LANES, LANES)] = ds[j] * rstd


def _make_kernel(rows, ws_rows, batch):
    r_per_w = rows // NW
    chunk = 128
    n_chunks = r_per_w // chunk
    ws_per_w = ws_rows // NW

    mesh = plsc.VectorSubcoreMesh(core_axis_name="c", subcore_axis_name="s",
                                  num_cores=NC, num_subcores=NS)

    @functools.partial(
        pl.kernel,
        out_type=(
            jax.ShapeDtypeStruct((batch, ws_rows, HIDDEN), jnp.float32),
            jax.ShapeDtypeStruct((rows, HIDDEN), jnp.float32),
        ),
        mesh=mesh,
        compiler_params=pltpu.CompilerParams(needs_layout_passes=False),
        scratch_types=[
            pltpu.VMEM((n_chunks, chunk), jnp.int32),
            pltpu.VMEM((n_chunks, chunk, HIDDEN), jnp.float32),
            pltpu.VMEM((ws_per_w, HIDDEN), jnp.float32),
            pltpu.SemaphoreType.DMA,
        ],
    )
    def kern(ids_hbm, ws_hbm, table_hbm, ws_out, emb_out, idx_v, rows_v,
             ws_v, sem):
        wid = lax.axis_index("s") * NC + lax.axis_index("c")
        base = wid * r_per_w

        for ci in range(n_chunks):
            pltpu.sync_copy(ids_hbm.at[pl.ds(base + ci * chunk, chunk)],
                            idx_v.at[ci])
        for ci in range(n_chunks):
            pltpu.async_copy(table_hbm.at[idx_v.at[ci]], rows_v.at[ci],
                             sem).wait()

        def body(r, _):
            for ci in range(n_chunks):
                _normalize_row(rows_v.at[ci], r)
            return _

        lax.fori_loop(0, chunk, body, None)

        for ci in range(n_chunks):
            pltpu.sync_copy(rows_v.at[ci],
                            emb_out.at[pl.ds(base + ci * chunk, chunk)])

        wbase = wid * ws_per_w
        pltpu.sync_copy(ws_hbm.at[pl.ds(wbase, ws_per_w)], ws_v)
        for r in range(ws_per_w):
            _normalize_row(ws_v, r)
        for b in range(batch):
            pltpu.sync_copy(ws_v, ws_out.at[b, pl.ds(wbase, ws_per_w)])

    return kern


def kernel(input_ids, attention_mask, init_workspace, emb_table):
    del attention_mask
    bs, seq = input_ids.shape
    ws_rows = init_workspace.shape[1]
    ids_flat = input_ids.reshape(bs * seq).astype(jnp.int32)
    ws2d = init_workspace.reshape(ws_rows, HIDDEN)
    kern = _make_kernel(bs * seq, ws_rows, bs)
    ws_out, emb_out = kern(ids_flat, ws2d, emb_table)
    return ws_out, emb_out.reshape(bs, seq, HIDDEN)
